# Initial kernel scaffold; baseline (speedup 1.0000x reference)
import jax
import jax.numpy as jnp
from jax.experimental import pallas as pl


def kernel(word_input, pose_input, word_table, pose_table):
    raise NotImplementedError("write your pallas kernel here")


# speedup vs baseline: 1.7523x; 1.7523x over previous
import jax
import jax.numpy as jnp
from jax import lax
from jax.experimental import pallas as pl
from jax.experimental.pallas import tpu as pltpu
from jax.experimental.pallas import tpu_sc as plsc

W_DIM = 64
P_DIM = 16
OUT_DIM = W_DIM + P_DIM
NC = 2
NS = 16
NW = NC * NS
CHUNK = 128


def kernel(word_input, pose_input, word_table, pose_table):
    B, S = word_input.shape
    n = B * S
    per_w = n // NW
    n_chunks = per_w // CHUNK
    wi = word_input.reshape(n)
    pi = pose_input.reshape(n)

    mesh = plsc.VectorSubcoreMesh(
        core_axis_name="core", subcore_axis_name="subcore"
    )

    @jax.jit
    def run(wt, pt, wi, pi):
        @pl.kernel(
            out_type=(
                jax.ShapeDtypeStruct((n, W_DIM), jnp.float32),
                jax.ShapeDtypeStruct((n, P_DIM), jnp.float32),
            ),
            mesh=mesh,
            scratch_types=[
                pltpu.VMEM((CHUNK,), jnp.int32),
                pltpu.VMEM((CHUNK,), jnp.int32),
                pltpu.VMEM((CHUNK, W_DIM), jnp.float32),
                pltpu.VMEM((CHUNK, P_DIM), jnp.float32),
                pltpu.SemaphoreType.DMA,
                pltpu.SemaphoreType.DMA,
            ],
            compiler_params=pltpu.CompilerParams(use_tc_tiling_on_sc=False),
        )
        def embed_kernel(
            wt_hbm, pt_hbm, wi_hbm, pi_hbm, ow_hbm, op_hbm,
            wi_v, pi_v, wrows_v, prows_v, sem_w, sem_p,
        ):
            wid = lax.axis_index("subcore") * NC + lax.axis_index("core")
            base = wid * per_w

            @pl.loop(0, n_chunks)
            def _(c):
                off = base + c * CHUNK
                pltpu.sync_copy(wi_hbm.at[pl.ds(off, CHUNK)], wi_v)
                pltpu.sync_copy(pi_hbm.at[pl.ds(off, CHUNK)], pi_v)
                cw = pltpu.async_copy(wt_hbm.at[wi_v], wrows_v, sem_w)
                cp = pltpu.async_copy(pt_hbm.at[pi_v], prows_v, sem_p)
                cw.wait()
                cp.wait()
                pltpu.sync_copy(wrows_v, ow_hbm.at[pl.ds(off, CHUNK)])
                pltpu.sync_copy(prows_v, op_hbm.at[pl.ds(off, CHUNK)])

        return embed_kernel(wt, pt, wi, pi)

    ow, op = run(word_table, pose_table, wi, pi)
    return jnp.concatenate(
        [ow.reshape(B, S, W_DIM), op.reshape(B, S, P_DIM)], axis=-1
    )


# speedup vs baseline: 1.9569x; 1.1168x over previous
import jax
import jax.numpy as jnp
from jax import lax
from jax.experimental import pallas as pl
from jax.experimental.pallas import tpu as pltpu
from jax.experimental.pallas import tpu_sc as plsc

W_DIM = 64
P_DIM = 16
OUT_DIM = W_DIM + P_DIM
NC = 2
NS = 16
NW = NC * NS
CHUNK = 128


def kernel(word_input, pose_input, word_table, pose_table):
    B, S = word_input.shape
    n = B * S
    per_w = n // NW
    n_chunks = per_w // CHUNK
    wi = word_input.reshape(n)
    pi = pose_input.reshape(n)

    mesh = plsc.VectorSubcoreMesh(
        core_axis_name="core", subcore_axis_name="subcore"
    )

    @jax.jit
    def run(wt, pt, wi, pi):
        @pl.kernel(
            out_type=jax.ShapeDtypeStruct((n, OUT_DIM), jnp.float32),
            mesh=mesh,
            scratch_types=[
                pltpu.VMEM((CHUNK,), jnp.int32),
                pltpu.VMEM((CHUNK,), jnp.int32),
                pltpu.VMEM((CHUNK, W_DIM), jnp.float32),
                pltpu.VMEM((CHUNK, P_DIM), jnp.float32),
                pltpu.SemaphoreType.DMA,
            ],
            compiler_params=pltpu.CompilerParams(use_tc_tiling_on_sc=False),
        )
        def embed_kernel(
            wt_hbm, pt_hbm, wi_hbm, pi_hbm, o_hbm,
            wi_v, pi_v, wrows_v, prows_v, sem_g,
        ):
            wid = lax.axis_index("subcore") * NC + lax.axis_index("core")
            base = wid * per_w

            @pl.loop(0, n_chunks)
            def _(c):
                off = base + c * CHUNK
                pltpu.sync_copy(wi_hbm.at[pl.ds(off, CHUNK)], wi_v)
                pltpu.sync_copy(pi_hbm.at[pl.ds(off, CHUNK)], pi_v)
                cw = pltpu.async_copy(wt_hbm.at[wi_v], wrows_v, sem_g)
                cp = pltpu.async_copy(pt_hbm.at[pi_v], prows_v, sem_g)
                cw.wait()
                cp.wait()
                pltpu.sync_copy(
                    wrows_v, o_hbm.at[pl.ds(off, CHUNK), 0:W_DIM]
                )
                pltpu.sync_copy(
                    prows_v, o_hbm.at[pl.ds(off, CHUNK), W_DIM:OUT_DIM]
                )

        return embed_kernel(wt, pt, wi, pi)

    out = run(word_table, pose_table, wi, pi)
    return out.reshape(B, S, OUT_DIM)


# speedup vs baseline: 2.0649x; 1.0552x over previous
import jax
import jax.numpy as jnp
from jax import lax
from jax.experimental import pallas as pl
from jax.experimental.pallas import tpu as pltpu
from jax.experimental.pallas import tpu_sc as plsc

W_DIM = 64
P_DIM = 16
OUT_DIM = W_DIM + P_DIM
NC = 2
NS = 16
NW = NC * NS
GW = 128
G = 4
CHUNK = GW * G


def kernel(word_input, pose_input, word_table, pose_table):
    B, S = word_input.shape
    n = B * S
    per_w = n // NW
    n_chunks = per_w // CHUNK
    assert per_w % CHUNK == 0 and n_chunks % 2 == 0
    wi = word_input.reshape(n)
    pi = pose_input.reshape(n)

    mesh = plsc.VectorSubcoreMesh(
        core_axis_name="core", subcore_axis_name="subcore"
    )

    @jax.jit
    def run(wt, pt, wi, pi):
        @pl.kernel(
            out_type=jax.ShapeDtypeStruct((n, OUT_DIM), jnp.float32),
            mesh=mesh,
            scratch_types=[
                pltpu.VMEM((CHUNK,), jnp.int32),
                pltpu.VMEM((CHUNK,), jnp.int32),
                pltpu.VMEM((CHUNK,), jnp.int32),
                pltpu.VMEM((CHUNK,), jnp.int32),
                pltpu.VMEM((CHUNK, W_DIM), jnp.float32),
                pltpu.VMEM((CHUNK, W_DIM), jnp.float32),
                pltpu.VMEM((CHUNK, P_DIM), jnp.float32),
                pltpu.VMEM((CHUNK, P_DIM), jnp.float32),
                pltpu.SemaphoreType.DMA,
                pltpu.SemaphoreType.DMA,
                pltpu.SemaphoreType.DMA,
                pltpu.SemaphoreType.DMA,
                pltpu.SemaphoreType.DMA,
                pltpu.SemaphoreType.DMA,
            ],
            compiler_params=pltpu.CompilerParams(use_tc_tiling_on_sc=False),
        )
        def embed_kernel(
            wt_hbm, pt_hbm, wi_hbm, pi_hbm, o_hbm,
            wi0, wi1, pi0, pi1, wr0, wr1, pr0, pr1,
            si0, si1, sg0, sg1, so0, so1,
        ):
            wid = lax.axis_index("subcore") * NC + lax.axis_index("core")
            base = wid * per_w
            wiv = (wi0, wi1)
            piv = (pi0, pi1)
            wrv = (wr0, wr1)
            prv = (pr0, pr1)
            si = (si0, si1)
            sg = (sg0, sg1)
            so = (so0, so1)

            def issue_idx(c, b):
                off = base + c * CHUNK
                pltpu.make_async_copy(
                    wi_hbm.at[pl.ds(off, CHUNK)], wiv[b], si[b]
                ).start()
                pltpu.make_async_copy(
                    pi_hbm.at[pl.ds(off, CHUNK)], piv[b], si[b]
                ).start()

            def wait_idx(b):
                pltpu.make_async_copy(
                    wi_hbm.at[pl.ds(base, CHUNK)], wiv[b], si[b]
                ).wait()
                pltpu.make_async_copy(
                    pi_hbm.at[pl.ds(base, CHUNK)], piv[b], si[b]
                ).wait()

            def issue_gathers(b):
                for g in range(G):
                    sl = pl.ds(g * GW, GW)
                    pltpu.make_async_copy(
                        wt_hbm.at[wiv[b].at[sl]], wrv[b].at[sl], sg[b]
                    ).start()
                    pltpu.make_async_copy(
                        pt_hbm.at[piv[b].at[sl]], prv[b].at[sl], sg[b]
                    ).start()

            def wait_gathers(b):
                pltpu.make_async_copy(
                    wt_hbm.at[pl.ds(0, CHUNK)], wrv[b], sg[b]
                ).wait()
                pltpu.make_async_copy(
                    wt_hbm.at[pl.ds(0, CHUNK), 0:P_DIM], prv[b], sg[b]
                ).wait()

            def issue_out(c, b):
                off = base + c * CHUNK
                pltpu.make_async_copy(
                    wrv[b], o_hbm.at[pl.ds(off, CHUNK), 0:W_DIM], so[b]
                ).start()
                pltpu.make_async_copy(
                    prv[b], o_hbm.at[pl.ds(off, CHUNK), W_DIM:OUT_DIM], so[b]
                ).start()

            def wait_out(b):
                pltpu.make_async_copy(
                    wrv[b], o_hbm.at[pl.ds(base, CHUNK), 0:W_DIM], so[b]
                ).wait()
                pltpu.make_async_copy(
                    prv[b], o_hbm.at[pl.ds(base, CHUNK), W_DIM:OUT_DIM], so[b]
                ).wait()

            issue_idx(0, 0)

            @pl.loop(0, n_chunks // 2)
            def _(ci):
                for j in (0, 1):
                    b = j
                    c = ci * 2 + j
                    wait_idx(b)
                    if j == 0:
                        @pl.when(ci >= 1)
                        def _():
                            wait_out(b)
                    else:
                        @pl.when(ci >= 1)
                        def _():
                            wait_out(b)
                    issue_gathers(b)
                    if j == 0:
                        @pl.when(ci >= 1)
                        def _():
                            wait_gathers(1)
                            issue_out(c - 1, 1)
                    else:
                        wait_gathers(0)
                        issue_out(c - 1, 0)
                    if j == 0:
                        issue_idx(c + 1, 1)
                    else:
                        @pl.when(ci < n_chunks // 2 - 1)
                        def _():
                            issue_idx(c + 1, 0)

            wait_gathers(1)
            issue_out(n_chunks - 1, 1)
            wait_out(0)
            wait_out(1)

        return embed_kernel(wt, pt, wi, pi)

    out = run(word_table, pose_table, wi, pi)
    return out.reshape(B, S, OUT_DIM)


# speedup vs baseline: 2.0661x; 1.0005x over previous
import jax
import jax.numpy as jnp
from jax import lax
from jax.experimental import pallas as pl
from jax.experimental.pallas import tpu as pltpu
from jax.experimental.pallas import tpu_sc as plsc

W_DIM = 64
P_DIM = 16
OUT_DIM = W_DIM + P_DIM
NC = 2
NS = 16
NW = NC * NS
BPC = 2
GSPLIT = ((0, 128), (128, 72), (200, 128), (328, 72))


def kernel(word_input, pose_input, word_table, pose_table):
    B, S = word_input.shape
    n = B * S
    chunk = BPC * S
    b_per_w = B // NW
    n_chunks = b_per_w // BPC
    wi = word_input.reshape(n)
    pi = pose_input.reshape(n)

    mesh = plsc.VectorSubcoreMesh(
        core_axis_name="core", subcore_axis_name="subcore"
    )

    @jax.jit
    def run(wt, pt, wi, pi):
        @pl.kernel(
            out_type=jax.ShapeDtypeStruct((B, S, OUT_DIM), jnp.float32),
            mesh=mesh,
            scratch_types=[
                pltpu.VMEM((chunk,), jnp.int32),
                pltpu.VMEM((chunk,), jnp.int32),
                pltpu.VMEM((chunk,), jnp.int32),
                pltpu.VMEM((chunk,), jnp.int32),
                pltpu.VMEM((BPC, S, W_DIM), jnp.float32),
                pltpu.VMEM((BPC, S, W_DIM), jnp.float32),
                pltpu.VMEM((BPC, S, P_DIM), jnp.float32),
                pltpu.VMEM((BPC, S, P_DIM), jnp.float32),
                pltpu.SemaphoreType.DMA,
                pltpu.SemaphoreType.DMA,
                pltpu.SemaphoreType.DMA,
                pltpu.SemaphoreType.DMA,
                pltpu.SemaphoreType.DMA,
                pltpu.SemaphoreType.DMA,
            ],
            compiler_params=pltpu.CompilerParams(use_tc_tiling_on_sc=False),
        )
        def embed_kernel(
            wt_hbm, pt_hbm, wi_hbm, pi_hbm, o_hbm,
            wi0, wi1, pi0, pi1, wr0, wr1, pr0, pr1,
            si0, si1, sg0, sg1, so0, so1,
        ):
            wid = lax.axis_index("subcore") * NC + lax.axis_index("core")
            base = wid * b_per_w * S
            bbase = wid * b_per_w
            wiv = (wi0, wi1)
            piv = (pi0, pi1)
            wrv = (wr0, wr1)
            prv = (pr0, pr1)
            si = (si0, si1)
            sg = (sg0, sg1)
            so = (so0, so1)

            def issue_idx(c, b):
                off = base + c * chunk
                pltpu.make_async_copy(
                    wi_hbm.at[pl.ds(off, chunk)], wiv[b], si[b]
                ).start()
                pltpu.make_async_copy(
                    pi_hbm.at[pl.ds(off, chunk)], piv[b], si[b]
                ).start()

            def wait_idx(b):
                pltpu.make_async_copy(
                    wi_hbm.at[pl.ds(base, chunk)], wiv[b], si[b]
                ).wait()
                pltpu.make_async_copy(
                    pi_hbm.at[pl.ds(base, chunk)], piv[b], si[b]
                ).wait()

            def issue_gathers(b):
                for g, (off, ln) in enumerate(GSPLIT):
                    bl = off // S
                    so_ = off % S
                    pltpu.make_async_copy(
                        wt_hbm.at[wiv[b].at[pl.ds(off, ln)]],
                        wrv[b].at[bl, pl.ds(so_, ln)],
                        sg[b],
                    ).start()
                    pltpu.make_async_copy(
                        pt_hbm.at[piv[b].at[pl.ds(off, ln)]],
                        prv[b].at[bl, pl.ds(so_, ln)],
                        sg[b],
                    ).start()

            def wait_gathers(b):
                pltpu.make_async_copy(
                    o_hbm.at[pl.ds(bbase, BPC), :, 0:W_DIM], wrv[b], sg[b]
                ).wait()
                pltpu.make_async_copy(
                    o_hbm.at[pl.ds(bbase, BPC), :, W_DIM:OUT_DIM],
                    prv[b],
                    sg[b],
                ).wait()

            def issue_out(c, b):
                bo = bbase + c * BPC
                pltpu.make_async_copy(
                    wrv[b], o_hbm.at[pl.ds(bo, BPC), :, 0:W_DIM], so[b]
                ).start()
                pltpu.make_async_copy(
                    prv[b], o_hbm.at[pl.ds(bo, BPC), :, W_DIM:OUT_DIM], so[b]
                ).start()

            def wait_out(b):
                pltpu.make_async_copy(
                    wrv[b], o_hbm.at[pl.ds(bbase, BPC), :, 0:W_DIM], so[b]
                ).wait()
                pltpu.make_async_copy(
                    prv[b], o_hbm.at[pl.ds(bbase, BPC), :, W_DIM:OUT_DIM],
                    so[b],
                ).wait()

            issue_idx(0, 0)

            @pl.loop(0, n_chunks // 2)
            def _(ci):
                for j in (0, 1):
                    b = j
                    c = ci * 2 + j
                    wait_idx(b)

                    @pl.when(ci >= 1)
                    def _():
                        wait_out(b)

                    issue_gathers(b)
                    if j == 0:
                        @pl.when(ci >= 1)
                        def _():
                            wait_gathers(1)
                            issue_out(c - 1, 1)
                        issue_idx(c + 1, 1)
                    else:
                        wait_gathers(0)
                        issue_out(c - 1, 0)

                        @pl.when(ci < n_chunks // 2 - 1)
                        def _():
                            issue_idx(c + 1, 0)

            wait_gathers(1)
            issue_out(n_chunks - 1, 1)
            wait_out(0)
            wait_out(1)

        return embed_kernel(wt, pt, wi, pi)

    return run(word_table, pose_table, wi, pi)
